# Initial kernel scaffold; baseline (speedup 1.0000x reference)
import functools

import jax
import jax.numpy as jnp
from jax import lax
from jax.experimental import pallas as pl
from jax.experimental.pallas import tpu as pltpu
from jax.experimental.pallas import tpu_sc as plsc

N = 10000
E = 320000
D = 128
H = 8
DH = 16
ALPHA = 0.2

NC = 2
NS = 16
CH = 128
NCHUNK = E // CH
CHUNK_PER_CORE = NCHUNK // NC
ROWS_PER_TILE = N // NS
ZROWS = 125

BLK = 1000
NB = N // BLK


def _edge_body(src_hbm, dst_hbm, h_hbm, fs_hbm, fd_hbm,
               num_out, den_out,
               idx_s, idx_d, bufh, bufm, bufs, bufd, bufatt,
               znum, zden, accn, accd, sem):
  c = lax.axis_index("c")
  s = lax.axis_index("s")

  def zero_loop(i, _):
    znum[pl.ds(i * 16, 16)] = jnp.zeros((16,), jnp.float32)
    return 0
  lax.fori_loop(0, (ZROWS * D) // 16, zero_loop, 0)

  def zero_loop2(i, _):
    zden[pl.ds(i * 16, 16)] = jnp.zeros((16,), jnp.float32)
    return 0
  lax.fori_loop(0, (ZROWS * DH) // 16, zero_loop2, 0)

  row0 = s * ROWS_PER_TILE
  for p in range(ROWS_PER_TILE // ZROWS):
    pltpu.sync_copy(znum.reshape(ZROWS, D),
                    accn.at[pl.ds(row0 + p * ZROWS, ZROWS)])
    pltpu.sync_copy(zden.reshape(ZROWS, DH),
                    accd.at[pl.ds(row0 + p * ZROWS, ZROWS)])
  plsc.subcore_barrier()

  base = c * CHUNK_PER_CORE + s * 78 + jnp.minimum(s, 2)
  nch = jnp.where(s < 2, 79, 78)

  def chunk_body(g, _):
    eoff = (base + g) * CH
    pltpu.sync_copy(src_hbm.at[pl.ds(eoff, CH)], idx_s)
    pltpu.sync_copy(dst_hbm.at[pl.ds(eoff, CH)], idx_d)
    cp_h = pltpu.async_copy(h_hbm.at[idx_s], bufh, sem)
    cp_s = pltpu.async_copy(fs_hbm.at[idx_s], bufs, sem)
    cp_d = pltpu.async_copy(fd_hbm.at[idx_d], bufd, sem)
    cp_h.wait()
    cp_s.wait()
    cp_d.wait()

    def edge(j, _):
      v = bufs[j] + bufd[j]
      att = jnp.exp(jnp.where(v >= 0.0, v, v * ALPHA))
      bufatt[j] = att
      jrow = jnp.full((16,), j, jnp.int32)
      for h in range(H):
        b = plsc.load_gather(bufatt, [jrow, jnp.full((16,), h, jnp.int32)])
        bufm[j, pl.ds(h * DH, DH)] = bufh[j, pl.ds(h * DH, DH)] * b
      return 0
    lax.fori_loop(0, CH, edge, 0)

    pltpu.sync_copy(bufm, accn.at[idx_d], add=True)
    pltpu.sync_copy(bufatt, accd.at[idx_d], add=True)
    return 0
  lax.fori_loop(0, nch, chunk_body, 0)

  plsc.subcore_barrier()

  out_row0 = c * N + s * ROWS_PER_TILE
  for p in range(ROWS_PER_TILE // ZROWS):
    pltpu.sync_copy(accn.at[pl.ds(row0 + p * ZROWS, ZROWS)],
                    num_out.at[pl.ds(out_row0 + p * ZROWS, ZROWS)])
    pltpu.sync_copy(accd.at[pl.ds(row0 + p * ZROWS, ZROWS)],
                    den_out.at[pl.ds(out_row0 + p * ZROWS, ZROWS)])


_edge_pass = functools.partial(
    pl.kernel,
    out_type=(
        jax.ShapeDtypeStruct((NC * N, D), jnp.float32),
        jax.ShapeDtypeStruct((NC * N, DH), jnp.float32),
    ),
    mesh=plsc.VectorSubcoreMesh(core_axis_name="c", subcore_axis_name="s"),
    scratch_types=[
        pltpu.VMEM((CH,), jnp.int32),
        pltpu.VMEM((CH,), jnp.int32),
        pltpu.VMEM((CH, D), jnp.float32),
        pltpu.VMEM((CH, D), jnp.float32),
        pltpu.VMEM((CH, DH), jnp.float32),
        pltpu.VMEM((CH, DH), jnp.float32),
        pltpu.VMEM((CH, DH), jnp.float32),
        pltpu.VMEM((ZROWS * D,), jnp.float32),
        pltpu.VMEM((ZROWS * DH,), jnp.float32),
        pltpu.VMEM_SHARED((N, D), jnp.float32),
        pltpu.VMEM_SHARED((N, DH), jnp.float32),
        pltpu.SemaphoreType.DMA,
    ],
)(_edge_body)


def _proj_body(x_ref, w_ref, ws_ref, wd_ref, h_ref, fs_ref, fd_ref):
  xb = x_ref[...]
  h_ref[...] = jnp.dot(xb, w_ref[...], preferred_element_type=jnp.float32)
  fs_ref[...] = jnp.dot(xb, ws_ref[...], preferred_element_type=jnp.float32)
  fd_ref[...] = jnp.dot(xb, wd_ref[...], preferred_element_type=jnp.float32)


def _proj(x, w, ws, wd):
  return pl.pallas_call(
      _proj_body,
      grid=(NB,),
      in_specs=[
          pl.BlockSpec((BLK, D), lambda i: (i, 0)),
          pl.BlockSpec((D, D), lambda i: (0, 0)),
          pl.BlockSpec((D, DH), lambda i: (0, 0)),
          pl.BlockSpec((D, DH), lambda i: (0, 0)),
      ],
      out_specs=[
          pl.BlockSpec((BLK, D), lambda i: (i, 0)),
          pl.BlockSpec((BLK, DH), lambda i: (i, 0)),
          pl.BlockSpec((BLK, DH), lambda i: (i, 0)),
      ],
      out_shape=[
          jax.ShapeDtypeStruct((N, D), jnp.float32),
          jax.ShapeDtypeStruct((N, DH), jnp.float32),
          jax.ShapeDtypeStruct((N, DH), jnp.float32),
      ],
  )(x, w, ws, wd)


def _head_expand():
  row = lax.broadcasted_iota(jnp.int32, (DH, D), 0)
  col = lax.broadcasted_iota(jnp.int32, (DH, D), 1)
  return (row == col // DH).astype(jnp.float32)


def _normalize(n0, n1, d0, d1):
  num = n0 + n1
  den = jnp.dot(d0 + d1, _head_expand(), preferred_element_type=jnp.float32)
  return num / (den + 1e-16)


def _mid_body(n0_ref, n1_ref, d0_ref, d1_ref, w_ref, ws_ref, wd_ref,
              h_ref, fs_ref, fd_ref):
  r = _normalize(n0_ref[...], n1_ref[...], d0_ref[...], d1_ref[...])
  hcat = jnp.where(r > 0.0, r, jnp.expm1(r))
  h_ref[...] = jnp.dot(hcat, w_ref[...], preferred_element_type=jnp.float32)
  fs_ref[...] = jnp.dot(hcat, ws_ref[...], preferred_element_type=jnp.float32)
  fd_ref[...] = jnp.dot(hcat, wd_ref[...], preferred_element_type=jnp.float32)


def _mid(num, den, w, ws, wd):
  return pl.pallas_call(
      _mid_body,
      grid=(NB,),
      in_specs=[
          pl.BlockSpec((BLK, D), lambda i: (i, 0)),
          pl.BlockSpec((BLK, D), lambda i: (i + NB, 0)),
          pl.BlockSpec((BLK, DH), lambda i: (i, 0)),
          pl.BlockSpec((BLK, DH), lambda i: (i + NB, 0)),
          pl.BlockSpec((D, D), lambda i: (0, 0)),
          pl.BlockSpec((D, DH), lambda i: (0, 0)),
          pl.BlockSpec((D, DH), lambda i: (0, 0)),
      ],
      out_specs=[
          pl.BlockSpec((BLK, D), lambda i: (i, 0)),
          pl.BlockSpec((BLK, DH), lambda i: (i, 0)),
          pl.BlockSpec((BLK, DH), lambda i: (i, 0)),
      ],
      out_shape=[
          jax.ShapeDtypeStruct((N, D), jnp.float32),
          jax.ShapeDtypeStruct((N, DH), jnp.float32),
          jax.ShapeDtypeStruct((N, DH), jnp.float32),
      ],
  )(num, num, den, den, w, ws, wd)


def _final_body(n0_ref, n1_ref, d0_ref, d1_ref, o_ref):
  r = _normalize(n0_ref[...], n1_ref[...], d0_ref[...], d1_ref[...])
  row = lax.broadcasted_iota(jnp.int32, (D, DH), 0)
  col = lax.broadcasted_iota(jnp.int32, (D, DH), 1)
  t = (row % DH == col).astype(jnp.float32) * (1.0 / H)
  o_ref[...] = jnp.dot(r, t, preferred_element_type=jnp.float32)


def _final(num, den):
  return pl.pallas_call(
      _final_body,
      grid=(NB,),
      in_specs=[
          pl.BlockSpec((BLK, D), lambda i: (i, 0)),
          pl.BlockSpec((BLK, D), lambda i: (i + NB, 0)),
          pl.BlockSpec((BLK, DH), lambda i: (i, 0)),
          pl.BlockSpec((BLK, DH), lambda i: (i + NB, 0)),
      ],
      out_specs=pl.BlockSpec((BLK, DH), lambda i: (i, 0)),
      out_shape=jax.ShapeDtypeStruct((N, DH), jnp.float32),
  )(num, num, den, den)


def _prep_weights(W, a):
  wf = W.transpose(1, 0, 2).reshape(W.shape[1], D)
  ws = jnp.einsum('hdk,hk->dh', W, a[:, :DH])
  wd = jnp.einsum('hdk,hk->dh', W, a[:, DH:])
  pad = jnp.zeros((W.shape[1], DH - H), jnp.float32)
  return wf, jnp.concatenate([ws, pad], 1), jnp.concatenate([wd, pad], 1)


def kernel(x, adj, W1, a1, W2, a2):
  src = adj[0]
  dst = adj[1]
  w1f, ws1, wd1 = _prep_weights(W1, a1)
  w2f, ws2, wd2 = _prep_weights(W2, a2)

  h1, fs1, fd1 = _proj(x, w1f, ws1, wd1)
  num1, den1 = _edge_pass(src, dst, h1, fs1, fd1)
  h2, fs2, fd2 = _mid(num1, den1, w2f, ws2, wd2)
  num2, den2 = _edge_pass(src, dst, h2, fs2, fd2)
  return _final(num2, den2)


# speedup vs baseline: 45.9537x; 45.9537x over previous
import functools

import jax
import jax.numpy as jnp
from jax import lax
from jax.experimental import pallas as pl
from jax.experimental.pallas import tpu as pltpu
from jax.experimental.pallas import tpu_sc as plsc

N = 10000
E = 320000
D = 128
H = 8
DH = 16
ALPHA = 0.2

NC = 2
NS = 16
CH = 128
NCHUNK = E // CH
CHUNK_PER_CORE = NCHUNK // NC
ROWS_PER_TILE = N // NS
ZROWS = 125
ZBROWS = 25

BLK = 1000
NB = N // BLK


def _edge_body(src_hbm, dst_hbm, h_hbm, fs_hbm, fd_hbm,
               num_out, den_out,
               idx_s, idx_d, bufh, bufs, bufd, bufatt,
               znum, zden, accn, accd, sem):
  c = lax.axis_index("c")
  s = lax.axis_index("s")

  def zero_loop(i, _):
    for q in range(D // 16):
      znum[i, pl.ds(q * 16, 16)] = jnp.zeros((16,), jnp.float32)
    zden[i] = jnp.zeros((16,), jnp.float32)
    return 0
  lax.fori_loop(0, ZBROWS, zero_loop, 0)

  row0 = s * ROWS_PER_TILE
  for p in range(ROWS_PER_TILE // ZBROWS):
    pltpu.sync_copy(znum, accn.at[pl.ds(row0 + p * ZBROWS, ZBROWS)])
    pltpu.sync_copy(zden, accd.at[pl.ds(row0 + p * ZBROWS, ZBROWS)])
  plsc.subcore_barrier()

  base = c * CHUNK_PER_CORE + s * 78 + jnp.minimum(s, 2)
  nch = jnp.where(s < 2, 79, 78)

  def chunk_body(g, _):
    eoff = (base + g) * CH
    pltpu.sync_copy(src_hbm.at[pl.ds(eoff, CH)], idx_s)
    pltpu.sync_copy(dst_hbm.at[pl.ds(eoff, CH)], idx_d)
    cp_h = pltpu.async_copy(h_hbm.at[idx_s], bufh, sem)
    cp_s = pltpu.async_copy(fs_hbm.at[idx_s], bufs, sem)
    cp_d = pltpu.async_copy(fd_hbm.at[idx_d], bufd, sem)
    cp_h.wait()
    cp_s.wait()
    cp_d.wait()

    def edge(j, _):
      v = bufs[j] + bufd[j]
      att = jnp.exp(jnp.where(v >= 0.0, v, v * ALPHA))
      bufatt[j] = att
      jrow = jnp.full((16,), j, jnp.int32)
      for h in range(H):
        b = plsc.load_gather(bufatt, [jrow, jnp.full((16,), h, jnp.int32)])
        bufh[j, pl.ds(h * DH, DH)] = bufh[j, pl.ds(h * DH, DH)] * b
      return 0
    lax.fori_loop(0, CH, edge, 0)

    pltpu.sync_copy(bufh, accn.at[idx_d], add=True)
    pltpu.sync_copy(bufatt, accd.at[idx_d], add=True)
    return 0
  lax.fori_loop(0, nch, chunk_body, 0)

  plsc.subcore_barrier()

  out_row0 = c * N + s * ROWS_PER_TILE
  for p in range(ROWS_PER_TILE // ZROWS):
    pltpu.sync_copy(accn.at[pl.ds(row0 + p * ZROWS, ZROWS)],
                    num_out.at[pl.ds(out_row0 + p * ZROWS, ZROWS)])
    pltpu.sync_copy(accd.at[pl.ds(row0 + p * ZROWS, ZROWS)],
                    den_out.at[pl.ds(out_row0 + p * ZROWS, ZROWS)])


_edge_pass = functools.partial(
    pl.kernel,
    out_type=(
        jax.ShapeDtypeStruct((NC * N, D), jnp.float32),
        jax.ShapeDtypeStruct((NC * N, DH), jnp.float32),
    ),
    mesh=plsc.VectorSubcoreMesh(core_axis_name="c", subcore_axis_name="s"),
    scratch_types=[
        pltpu.VMEM((CH,), jnp.int32),
        pltpu.VMEM((CH,), jnp.int32),
        pltpu.VMEM((CH, D), jnp.float32),
        pltpu.VMEM((CH, DH), jnp.float32),
        pltpu.VMEM((CH, DH), jnp.float32),
        pltpu.VMEM((CH, DH), jnp.float32),
        pltpu.VMEM((ZBROWS, D), jnp.float32),
        pltpu.VMEM((ZBROWS, DH), jnp.float32),
        pltpu.VMEM_SHARED((N, D), jnp.float32),
        pltpu.VMEM_SHARED((N, DH), jnp.float32),
        pltpu.SemaphoreType.DMA,
    ],
    compiler_params=pltpu.CompilerParams(
        use_tc_tiling_on_sc=False, needs_layout_passes=False),
)(_edge_body)


def _proj_body(x_ref, w_ref, ws_ref, wd_ref, h_ref, fs_ref, fd_ref):
  xb = x_ref[...]
  h_ref[...] = jnp.dot(xb, w_ref[...], preferred_element_type=jnp.float32)
  fs_ref[...] = jnp.dot(xb, ws_ref[...], preferred_element_type=jnp.float32)
  fd_ref[...] = jnp.dot(xb, wd_ref[...], preferred_element_type=jnp.float32)


def _proj(x, w, ws, wd):
  return pl.pallas_call(
      _proj_body,
      grid=(NB,),
      in_specs=[
          pl.BlockSpec((BLK, D), lambda i: (i, 0)),
          pl.BlockSpec((D, D), lambda i: (0, 0)),
          pl.BlockSpec((D, DH), lambda i: (0, 0)),
          pl.BlockSpec((D, DH), lambda i: (0, 0)),
      ],
      out_specs=[
          pl.BlockSpec((BLK, D), lambda i: (i, 0)),
          pl.BlockSpec((BLK, DH), lambda i: (i, 0)),
          pl.BlockSpec((BLK, DH), lambda i: (i, 0)),
      ],
      out_shape=[
          jax.ShapeDtypeStruct((N, D), jnp.float32),
          jax.ShapeDtypeStruct((N, DH), jnp.float32),
          jax.ShapeDtypeStruct((N, DH), jnp.float32),
      ],
  )(x, w, ws, wd)


def _head_expand():
  row = lax.broadcasted_iota(jnp.int32, (DH, D), 0)
  col = lax.broadcasted_iota(jnp.int32, (DH, D), 1)
  return (row == col // DH).astype(jnp.float32)


def _normalize(n0, n1, d0, d1):
  num = n0 + n1
  den = jnp.dot(d0 + d1, _head_expand(), preferred_element_type=jnp.float32)
  return num / (den + 1e-16)


def _mid_body(n0_ref, n1_ref, d0_ref, d1_ref, w_ref, ws_ref, wd_ref,
              h_ref, fs_ref, fd_ref):
  r = _normalize(n0_ref[...], n1_ref[...], d0_ref[...], d1_ref[...])
  hcat = jnp.where(r > 0.0, r, jnp.exp(jnp.minimum(r, 0.0)) - 1.0)
  h_ref[...] = jnp.dot(hcat, w_ref[...], preferred_element_type=jnp.float32)
  fs_ref[...] = jnp.dot(hcat, ws_ref[...], preferred_element_type=jnp.float32)
  fd_ref[...] = jnp.dot(hcat, wd_ref[...], preferred_element_type=jnp.float32)


def _mid(num, den, w, ws, wd):
  return pl.pallas_call(
      _mid_body,
      grid=(NB,),
      in_specs=[
          pl.BlockSpec((BLK, D), lambda i: (i, 0)),
          pl.BlockSpec((BLK, D), lambda i: (i + NB, 0)),
          pl.BlockSpec((BLK, DH), lambda i: (i, 0)),
          pl.BlockSpec((BLK, DH), lambda i: (i + NB, 0)),
          pl.BlockSpec((D, D), lambda i: (0, 0)),
          pl.BlockSpec((D, DH), lambda i: (0, 0)),
          pl.BlockSpec((D, DH), lambda i: (0, 0)),
      ],
      out_specs=[
          pl.BlockSpec((BLK, D), lambda i: (i, 0)),
          pl.BlockSpec((BLK, DH), lambda i: (i, 0)),
          pl.BlockSpec((BLK, DH), lambda i: (i, 0)),
      ],
      out_shape=[
          jax.ShapeDtypeStruct((N, D), jnp.float32),
          jax.ShapeDtypeStruct((N, DH), jnp.float32),
          jax.ShapeDtypeStruct((N, DH), jnp.float32),
      ],
  )(num, num, den, den, w, ws, wd)


def _final_body(n0_ref, n1_ref, d0_ref, d1_ref, o_ref):
  r = _normalize(n0_ref[...], n1_ref[...], d0_ref[...], d1_ref[...])
  row = lax.broadcasted_iota(jnp.int32, (D, DH), 0)
  col = lax.broadcasted_iota(jnp.int32, (D, DH), 1)
  t = (row % DH == col).astype(jnp.float32) * (1.0 / H)
  o_ref[...] = jnp.dot(r, t, preferred_element_type=jnp.float32)


def _final(num, den):
  return pl.pallas_call(
      _final_body,
      grid=(NB,),
      in_specs=[
          pl.BlockSpec((BLK, D), lambda i: (i, 0)),
          pl.BlockSpec((BLK, D), lambda i: (i + NB, 0)),
          pl.BlockSpec((BLK, DH), lambda i: (i, 0)),
          pl.BlockSpec((BLK, DH), lambda i: (i + NB, 0)),
      ],
      out_specs=pl.BlockSpec((BLK, DH), lambda i: (i, 0)),
      out_shape=jax.ShapeDtypeStruct((N, DH), jnp.float32),
  )(num, num, den, den)


def _prep_weights(W, a):
  wf = W.transpose(1, 0, 2).reshape(W.shape[1], D)
  ws = jnp.einsum('hdk,hk->dh', W, a[:, :DH])
  wd = jnp.einsum('hdk,hk->dh', W, a[:, DH:])
  pad = jnp.zeros((W.shape[1], DH - H), jnp.float32)
  return wf, jnp.concatenate([ws, pad], 1), jnp.concatenate([wd, pad], 1)


def kernel(x, adj, W1, a1, W2, a2):
  src = adj[0]
  dst = adj[1]
  w1f, ws1, wd1 = _prep_weights(W1, a1)
  w2f, ws2, wd2 = _prep_weights(W2, a2)

  h1, fs1, fd1 = _proj(x, w1f, ws1, wd1)
  num1, den1 = _edge_pass(src, dst, h1, fs1, fd1)
  h2, fs2, fd2 = _mid(num1, den1, w2f, ws2, wd2)
  num2, den2 = _edge_pass(src, dst, h2, fs2, fd2)
  return _final(num2, den2)


# speedup vs baseline: 126.8470x; 2.7603x over previous
import functools

import jax
import jax.numpy as jnp
from jax import lax
from jax.experimental import pallas as pl
from jax.experimental.pallas import tpu as pltpu
from jax.experimental.pallas import tpu_sc as plsc

N = 10000
E = 320000
D = 128
H = 8
DH = 16
ALPHA = 0.2

NC = 2
NS = 16
CH = 128
NCHUNK = E // CH
CHUNK_PER_CORE = NCHUNK // NC
ROWS_PER_TILE = N // NS
ZROWS = 125

BLK = 1000
NB = N // BLK


def _edge_body(src_hbm, dst_hbm, h_hbm, fs_hbm, fd_hbm,
               num_out, den_out,
               idxs0, idxd0, idxs1, idxd1,
               bufh0, bufh1, bufs, bufd, bufatt,
               accn, accd, sem_h, sem_f):
  c = lax.axis_index("c")
  s = lax.axis_index("s")

  def zero_loop(i, _):
    for q in range(D // 16):
      bufh0[i, pl.ds(q * 16, 16)] = jnp.zeros((16,), jnp.float32)
    bufs[i] = jnp.zeros((16,), jnp.float32)
    return 0
  lax.fori_loop(0, CH, zero_loop, 0)

  row0 = s * ROWS_PER_TILE
  for p in range(ROWS_PER_TILE // ZROWS):
    pltpu.sync_copy(bufh0.at[pl.ds(0, ZROWS)],
                    accn.at[pl.ds(row0 + p * ZROWS, ZROWS)])
    pltpu.sync_copy(bufs.at[pl.ds(0, ZROWS)],
                    accd.at[pl.ds(row0 + p * ZROWS, ZROWS)])
  plsc.subcore_barrier()

  def fetch_idx(e, is_, id_):
    off = e * CH
    pltpu.sync_copy(src_hbm.at[pl.ds(off, CH)], is_)
    pltpu.sync_copy(dst_hbm.at[pl.ds(off, CH)], id_)

  def fire_h(is_, bh):
    pltpu.async_copy(h_hbm.at[is_], bh, sem_h)

  def fire_f(is_, id_):
    pltpu.async_copy(fs_hbm.at[is_], bufs, sem_f)
    pltpu.async_copy(fd_hbm.at[id_], bufd, sem_f)

  def wait_h(is_, bh):
    pltpu.make_async_copy(h_hbm.at[is_], bh, sem_h).wait()

  def wait_f(is_, id_):
    pltpu.make_async_copy(fs_hbm.at[is_], bufs, sem_f).wait()
    pltpu.make_async_copy(fd_hbm.at[id_], bufd, sem_f).wait()

  def att_phase():
    @plsc.parallel_loop(0, CH, unroll=8)
    def _(j):
      v = bufs[j] + bufd[j]
      bufatt[j] = jnp.exp(jnp.where(v >= 0.0, v, v * ALPHA))

  def scale_phase(bh):
    @plsc.parallel_loop(0, CH, unroll=4)
    def _(j):
      jrow = jnp.full((16,), j, jnp.int32)
      for h in range(H):
        b = plsc.load_gather(bufatt, [jrow, jnp.full((16,), h, jnp.int32)])
        bh[j, pl.ds(h * DH, DH)] = bh[j, pl.ds(h * DH, DH)] * b

  def scatter(bh, id_):
    pltpu.sync_copy(bh, accn.at[id_], add=True)
    pltpu.sync_copy(bufatt, accd.at[id_], add=True)

  base = c * CHUNK_PER_CORE + s * 78

  fetch_idx(base, idxs0, idxd0)
  fire_f(idxs0, idxd0)
  fire_h(idxs0, bufh0)
  fetch_idx(base + 1, idxs1, idxd1)

  def pair(t, _):
    wait_f(idxs0, idxd0)
    wait_h(idxs0, bufh0)
    att_phase()
    fire_f(idxs1, idxd1)
    fire_h(idxs1, bufh1)
    scale_phase(bufh0)
    scatter(bufh0, idxd0)
    fetch_idx(base + 2 * t + 2, idxs0, idxd0)

    wait_f(idxs1, idxd1)
    wait_h(idxs1, bufh1)
    att_phase()

    @pl.when(t < (78 // 2) - 1)
    def _():
      fire_f(idxs0, idxd0)
      fire_h(idxs0, bufh0)

    scale_phase(bufh1)
    scatter(bufh1, idxd1)
    fetch_idx(base + 2 * t + 3, idxs1, idxd1)
    return 0
  lax.fori_loop(0, 78 // 2, pair, 0)

  @pl.when(s < CHUNK_PER_CORE - NS * 78)
  def _():
    fetch_idx(c * CHUNK_PER_CORE + NS * 78 + s, idxs0, idxd0)
    fire_f(idxs0, idxd0)
    fire_h(idxs0, bufh0)
    wait_f(idxs0, idxd0)
    wait_h(idxs0, bufh0)
    att_phase()
    scale_phase(bufh0)
    scatter(bufh0, idxd0)

  plsc.subcore_barrier()

  out_row0 = c * N + s * ROWS_PER_TILE
  for p in range(ROWS_PER_TILE // ZROWS):
    pltpu.sync_copy(accn.at[pl.ds(row0 + p * ZROWS, ZROWS)],
                    num_out.at[pl.ds(out_row0 + p * ZROWS, ZROWS)])
    pltpu.sync_copy(accd.at[pl.ds(row0 + p * ZROWS, ZROWS)],
                    den_out.at[pl.ds(out_row0 + p * ZROWS, ZROWS)])


_edge_pass = functools.partial(
    pl.kernel,
    out_type=(
        jax.ShapeDtypeStruct((NC * N, D), jnp.float32),
        jax.ShapeDtypeStruct((NC * N, DH), jnp.float32),
    ),
    mesh=plsc.VectorSubcoreMesh(core_axis_name="c", subcore_axis_name="s"),
    scratch_types=[
        pltpu.VMEM((CH,), jnp.int32),
        pltpu.VMEM((CH,), jnp.int32),
        pltpu.VMEM((CH,), jnp.int32),
        pltpu.VMEM((CH,), jnp.int32),
        pltpu.VMEM((CH, D), jnp.float32),
        pltpu.VMEM((CH, D), jnp.float32),
        pltpu.VMEM((CH, DH), jnp.float32),
        pltpu.VMEM((CH, DH), jnp.float32),
        pltpu.VMEM((CH, DH), jnp.float32),
        pltpu.VMEM_SHARED((N, D), jnp.float32),
        pltpu.VMEM_SHARED((N, DH), jnp.float32),
        pltpu.SemaphoreType.DMA,
        pltpu.SemaphoreType.DMA,
    ],
    compiler_params=pltpu.CompilerParams(
        use_tc_tiling_on_sc=False, needs_layout_passes=False),
)(_edge_body)


def _proj_body(x_ref, w_ref, ws_ref, wd_ref, h_ref, fs_ref, fd_ref):
  xb = x_ref[...]
  h_ref[...] = jnp.dot(xb, w_ref[...], preferred_element_type=jnp.float32)
  fs_ref[...] = jnp.dot(xb, ws_ref[...], preferred_element_type=jnp.float32)
  fd_ref[...] = jnp.dot(xb, wd_ref[...], preferred_element_type=jnp.float32)


def _proj(x, w, ws, wd):
  return pl.pallas_call(
      _proj_body,
      grid=(NB,),
      in_specs=[
          pl.BlockSpec((BLK, D), lambda i: (i, 0)),
          pl.BlockSpec((D, D), lambda i: (0, 0)),
          pl.BlockSpec((D, DH), lambda i: (0, 0)),
          pl.BlockSpec((D, DH), lambda i: (0, 0)),
      ],
      out_specs=[
          pl.BlockSpec((BLK, D), lambda i: (i, 0)),
          pl.BlockSpec((BLK, DH), lambda i: (i, 0)),
          pl.BlockSpec((BLK, DH), lambda i: (i, 0)),
      ],
      out_shape=[
          jax.ShapeDtypeStruct((N, D), jnp.float32),
          jax.ShapeDtypeStruct((N, DH), jnp.float32),
          jax.ShapeDtypeStruct((N, DH), jnp.float32),
      ],
  )(x, w, ws, wd)


def _head_expand():
  row = lax.broadcasted_iota(jnp.int32, (DH, D), 0)
  col = lax.broadcasted_iota(jnp.int32, (DH, D), 1)
  return (row == col // DH).astype(jnp.float32)


def _normalize(n0, n1, d0, d1):
  num = n0 + n1
  den = jnp.dot(d0 + d1, _head_expand(), preferred_element_type=jnp.float32)
  return num / (den + 1e-16)


def _mid_body(n0_ref, n1_ref, d0_ref, d1_ref, w_ref, ws_ref, wd_ref,
              h_ref, fs_ref, fd_ref):
  r = _normalize(n0_ref[...], n1_ref[...], d0_ref[...], d1_ref[...])
  hcat = jnp.where(r > 0.0, r, jnp.exp(jnp.minimum(r, 0.0)) - 1.0)
  h_ref[...] = jnp.dot(hcat, w_ref[...], preferred_element_type=jnp.float32)
  fs_ref[...] = jnp.dot(hcat, ws_ref[...], preferred_element_type=jnp.float32)
  fd_ref[...] = jnp.dot(hcat, wd_ref[...], preferred_element_type=jnp.float32)


def _mid(num, den, w, ws, wd):
  return pl.pallas_call(
      _mid_body,
      grid=(NB,),
      in_specs=[
          pl.BlockSpec((BLK, D), lambda i: (i, 0)),
          pl.BlockSpec((BLK, D), lambda i: (i + NB, 0)),
          pl.BlockSpec((BLK, DH), lambda i: (i, 0)),
          pl.BlockSpec((BLK, DH), lambda i: (i + NB, 0)),
          pl.BlockSpec((D, D), lambda i: (0, 0)),
          pl.BlockSpec((D, DH), lambda i: (0, 0)),
          pl.BlockSpec((D, DH), lambda i: (0, 0)),
      ],
      out_specs=[
          pl.BlockSpec((BLK, D), lambda i: (i, 0)),
          pl.BlockSpec((BLK, DH), lambda i: (i, 0)),
          pl.BlockSpec((BLK, DH), lambda i: (i, 0)),
      ],
      out_shape=[
          jax.ShapeDtypeStruct((N, D), jnp.float32),
          jax.ShapeDtypeStruct((N, DH), jnp.float32),
          jax.ShapeDtypeStruct((N, DH), jnp.float32),
      ],
  )(num, num, den, den, w, ws, wd)


def _final_body(n0_ref, n1_ref, d0_ref, d1_ref, o_ref):
  r = _normalize(n0_ref[...], n1_ref[...], d0_ref[...], d1_ref[...])
  row = lax.broadcasted_iota(jnp.int32, (D, DH), 0)
  col = lax.broadcasted_iota(jnp.int32, (D, DH), 1)
  t = (row % DH == col).astype(jnp.float32) * (1.0 / H)
  o_ref[...] = jnp.dot(r, t, preferred_element_type=jnp.float32)


def _final(num, den):
  return pl.pallas_call(
      _final_body,
      grid=(NB,),
      in_specs=[
          pl.BlockSpec((BLK, D), lambda i: (i, 0)),
          pl.BlockSpec((BLK, D), lambda i: (i + NB, 0)),
          pl.BlockSpec((BLK, DH), lambda i: (i, 0)),
          pl.BlockSpec((BLK, DH), lambda i: (i + NB, 0)),
      ],
      out_specs=pl.BlockSpec((BLK, DH), lambda i: (i, 0)),
      out_shape=jax.ShapeDtypeStruct((N, DH), jnp.float32),
  )(num, num, den, den)


def _prep_weights(W, a):
  wf = W.transpose(1, 0, 2).reshape(W.shape[1], D)
  ws = jnp.einsum('hdk,hk->dh', W, a[:, :DH])
  wd = jnp.einsum('hdk,hk->dh', W, a[:, DH:])
  pad = jnp.zeros((W.shape[1], DH - H), jnp.float32)
  return wf, jnp.concatenate([ws, pad], 1), jnp.concatenate([wd, pad], 1)


def kernel(x, adj, W1, a1, W2, a2):
  src = adj[0]
  dst = adj[1]
  w1f, ws1, wd1 = _prep_weights(W1, a1)
  w2f, ws2, wd2 = _prep_weights(W2, a2)

  h1, fs1, fd1 = _proj(x, w1f, ws1, wd1)
  num1, den1 = _edge_pass(src, dst, h1, fs1, fd1)
  h2, fs2, fd2 = _mid(num1, den1, w2f, ws2, wd2)
  num2, den2 = _edge_pass(src, dst, h2, fs2, fd2)
  return _final(num2, den2)


# speedup vs baseline: 128.8768x; 1.0160x over previous
import functools

import jax
import jax.numpy as jnp
from jax import lax
from jax.experimental import pallas as pl
from jax.experimental.pallas import tpu as pltpu
from jax.experimental.pallas import tpu_sc as plsc

N = 10000
E = 320000
D = 128
H = 8
DH = 16
ALPHA = 0.2

NC = 2
NS = 16
CH = 128
NCHUNK = E // CH
CHUNK_PER_CORE = NCHUNK // NC
ROWS_PER_TILE = N // NS
ZROWS = 125

BLK = 1000
NB = N // BLK


def _edge_body(src_hbm, dst_hbm, h_hbm, fs_hbm, fd_hbm,
               num_out, den_out,
               idxs0, idxd0, idxs1, idxd1,
               bufh0, bufh1, bufs, bufd, bufatt,
               accn, accd, sem_h, sem_f, sem_sc):
  c = lax.axis_index("c")
  s = lax.axis_index("s")

  def zero_loop(i, _):
    for q in range(D // 16):
      bufh0[i, pl.ds(q * 16, 16)] = jnp.zeros((16,), jnp.float32)
    bufs[i] = jnp.zeros((16,), jnp.float32)
    return 0
  lax.fori_loop(0, CH, zero_loop, 0)

  row0 = s * ROWS_PER_TILE
  for p in range(ROWS_PER_TILE // ZROWS):
    pltpu.sync_copy(bufh0.at[pl.ds(0, ZROWS)],
                    accn.at[pl.ds(row0 + p * ZROWS, ZROWS)])
    pltpu.sync_copy(bufs.at[pl.ds(0, ZROWS)],
                    accd.at[pl.ds(row0 + p * ZROWS, ZROWS)])
  plsc.subcore_barrier()

  def fetch_idx(e, is_, id_):
    off = e * CH
    pltpu.sync_copy(src_hbm.at[pl.ds(off, CH)], is_)
    pltpu.sync_copy(dst_hbm.at[pl.ds(off, CH)], id_)

  def fire_h(is_, bh):
    pltpu.async_copy(h_hbm.at[is_], bh, sem_h)

  def fire_f(is_, id_):
    pltpu.async_copy(fs_hbm.at[is_], bufs, sem_f)
    pltpu.async_copy(fd_hbm.at[id_], bufd, sem_f)

  def wait_h(is_, bh):
    pltpu.make_async_copy(h_hbm.at[is_], bh, sem_h).wait()

  def wait_f(is_, id_):
    pltpu.make_async_copy(fs_hbm.at[is_], bufs, sem_f).wait()
    pltpu.make_async_copy(fd_hbm.at[id_], bufd, sem_f).wait()

  def att_phase():
    @plsc.parallel_loop(0, CH, unroll=8)
    def _(j):
      v = bufs[j] + bufd[j]
      bufatt[j] = jnp.exp(jnp.where(v >= 0.0, v, v * ALPHA))

  def scale_phase(bh):
    @plsc.parallel_loop(0, CH, unroll=8)
    def _(j):
      jrow = jnp.full((16,), j, jnp.int32)
      for h in range(H):
        b = plsc.load_gather(bufatt, [jrow, jnp.full((16,), h, jnp.int32)])
        bh[j, pl.ds(h * DH, DH)] = bh[j, pl.ds(h * DH, DH)] * b

  def scatter_async(bh, id_):
    pltpu.async_copy(bh, accn.at[id_], sem_sc, add=True)
    pltpu.sync_copy(bufatt, accd.at[id_], add=True)

  def wait_sc(bh, id_):
    pltpu.make_async_copy(bh, accn.at[id_], sem_sc).wait()

  def scatter_sync(bh, id_):
    pltpu.sync_copy(bh, accn.at[id_], add=True)
    pltpu.sync_copy(bufatt, accd.at[id_], add=True)

  base = c * CHUNK_PER_CORE + s * 78

  fetch_idx(base, idxs0, idxd0)
  fire_f(idxs0, idxd0)
  fire_h(idxs0, bufh0)

  def pair(t, _):
    wait_f(idxs0, idxd0)
    wait_h(idxs0, bufh0)
    att_phase()

    @pl.when(t > 0)
    def _():
      wait_sc(bufh1, idxd1)

    fetch_idx(base + 2 * t + 1, idxs1, idxd1)
    fire_f(idxs1, idxd1)
    fire_h(idxs1, bufh1)
    scale_phase(bufh0)
    scatter_async(bufh0, idxd0)

    wait_f(idxs1, idxd1)
    wait_h(idxs1, bufh1)
    att_phase()
    wait_sc(bufh0, idxd0)
    fetch_idx(base + 2 * t + 2, idxs0, idxd0)

    @pl.when(t < (78 // 2) - 1)
    def _():
      fire_f(idxs0, idxd0)
      fire_h(idxs0, bufh0)

    scale_phase(bufh1)
    scatter_async(bufh1, idxd1)
    return 0
  lax.fori_loop(0, 78 // 2, pair, 0)
  wait_sc(bufh1, idxd1)

  @pl.when(s < CHUNK_PER_CORE - NS * 78)
  def _():
    fetch_idx(c * CHUNK_PER_CORE + NS * 78 + s, idxs0, idxd0)
    fire_f(idxs0, idxd0)
    fire_h(idxs0, bufh0)
    wait_f(idxs0, idxd0)
    wait_h(idxs0, bufh0)
    att_phase()
    scale_phase(bufh0)
    scatter_sync(bufh0, idxd0)

  plsc.subcore_barrier()

  out_row0 = c * N + s * ROWS_PER_TILE
  for p in range(ROWS_PER_TILE // ZROWS):
    pltpu.sync_copy(accn.at[pl.ds(row0 + p * ZROWS, ZROWS)],
                    num_out.at[pl.ds(out_row0 + p * ZROWS, ZROWS)])
    pltpu.sync_copy(accd.at[pl.ds(row0 + p * ZROWS, ZROWS)],
                    den_out.at[pl.ds(out_row0 + p * ZROWS, ZROWS)])


_edge_pass = functools.partial(
    pl.kernel,
    out_type=(
        jax.ShapeDtypeStruct((NC * N, D), jnp.float32),
        jax.ShapeDtypeStruct((NC * N, DH), jnp.float32),
    ),
    mesh=plsc.VectorSubcoreMesh(core_axis_name="c", subcore_axis_name="s"),
    scratch_types=[
        pltpu.VMEM((CH,), jnp.int32),
        pltpu.VMEM((CH,), jnp.int32),
        pltpu.VMEM((CH,), jnp.int32),
        pltpu.VMEM((CH,), jnp.int32),
        pltpu.VMEM((CH, D), jnp.float32),
        pltpu.VMEM((CH, D), jnp.float32),
        pltpu.VMEM((CH, DH), jnp.float32),
        pltpu.VMEM((CH, DH), jnp.float32),
        pltpu.VMEM((CH, DH), jnp.float32),
        pltpu.VMEM_SHARED((N, D), jnp.float32),
        pltpu.VMEM_SHARED((N, DH), jnp.float32),
        pltpu.SemaphoreType.DMA,
        pltpu.SemaphoreType.DMA,
        pltpu.SemaphoreType.DMA,
    ],
    compiler_params=pltpu.CompilerParams(
        use_tc_tiling_on_sc=False, needs_layout_passes=False),
)(_edge_body)


def _proj_body(x_ref, w_ref, ws_ref, wd_ref, h_ref, fs_ref, fd_ref):
  xb = x_ref[...]
  h_ref[...] = jnp.dot(xb, w_ref[...], preferred_element_type=jnp.float32)
  fs_ref[...] = jnp.dot(xb, ws_ref[...], preferred_element_type=jnp.float32)
  fd_ref[...] = jnp.dot(xb, wd_ref[...], preferred_element_type=jnp.float32)


def _proj(x, w, ws, wd):
  return pl.pallas_call(
      _proj_body,
      grid=(NB,),
      in_specs=[
          pl.BlockSpec((BLK, D), lambda i: (i, 0)),
          pl.BlockSpec((D, D), lambda i: (0, 0)),
          pl.BlockSpec((D, DH), lambda i: (0, 0)),
          pl.BlockSpec((D, DH), lambda i: (0, 0)),
      ],
      out_specs=[
          pl.BlockSpec((BLK, D), lambda i: (i, 0)),
          pl.BlockSpec((BLK, DH), lambda i: (i, 0)),
          pl.BlockSpec((BLK, DH), lambda i: (i, 0)),
      ],
      out_shape=[
          jax.ShapeDtypeStruct((N, D), jnp.float32),
          jax.ShapeDtypeStruct((N, DH), jnp.float32),
          jax.ShapeDtypeStruct((N, DH), jnp.float32),
      ],
  )(x, w, ws, wd)


def _head_expand():
  row = lax.broadcasted_iota(jnp.int32, (DH, D), 0)
  col = lax.broadcasted_iota(jnp.int32, (DH, D), 1)
  return (row == col // DH).astype(jnp.float32)


def _normalize(n0, n1, d0, d1):
  num = n0 + n1
  den = jnp.dot(d0 + d1, _head_expand(), preferred_element_type=jnp.float32)
  return num / (den + 1e-16)


def _mid_body(n0_ref, n1_ref, d0_ref, d1_ref, w_ref, ws_ref, wd_ref,
              h_ref, fs_ref, fd_ref):
  r = _normalize(n0_ref[...], n1_ref[...], d0_ref[...], d1_ref[...])
  hcat = jnp.where(r > 0.0, r, jnp.exp(jnp.minimum(r, 0.0)) - 1.0)
  h_ref[...] = jnp.dot(hcat, w_ref[...], preferred_element_type=jnp.float32)
  fs_ref[...] = jnp.dot(hcat, ws_ref[...], preferred_element_type=jnp.float32)
  fd_ref[...] = jnp.dot(hcat, wd_ref[...], preferred_element_type=jnp.float32)


def _mid(num, den, w, ws, wd):
  return pl.pallas_call(
      _mid_body,
      grid=(NB,),
      in_specs=[
          pl.BlockSpec((BLK, D), lambda i: (i, 0)),
          pl.BlockSpec((BLK, D), lambda i: (i + NB, 0)),
          pl.BlockSpec((BLK, DH), lambda i: (i, 0)),
          pl.BlockSpec((BLK, DH), lambda i: (i + NB, 0)),
          pl.BlockSpec((D, D), lambda i: (0, 0)),
          pl.BlockSpec((D, DH), lambda i: (0, 0)),
          pl.BlockSpec((D, DH), lambda i: (0, 0)),
      ],
      out_specs=[
          pl.BlockSpec((BLK, D), lambda i: (i, 0)),
          pl.BlockSpec((BLK, DH), lambda i: (i, 0)),
          pl.BlockSpec((BLK, DH), lambda i: (i, 0)),
      ],
      out_shape=[
          jax.ShapeDtypeStruct((N, D), jnp.float32),
          jax.ShapeDtypeStruct((N, DH), jnp.float32),
          jax.ShapeDtypeStruct((N, DH), jnp.float32),
      ],
  )(num, num, den, den, w, ws, wd)


def _final_body(n0_ref, n1_ref, d0_ref, d1_ref, o_ref):
  r = _normalize(n0_ref[...], n1_ref[...], d0_ref[...], d1_ref[...])
  row = lax.broadcasted_iota(jnp.int32, (D, DH), 0)
  col = lax.broadcasted_iota(jnp.int32, (D, DH), 1)
  t = (row % DH == col).astype(jnp.float32) * (1.0 / H)
  o_ref[...] = jnp.dot(r, t, preferred_element_type=jnp.float32)


def _final(num, den):
  return pl.pallas_call(
      _final_body,
      grid=(NB,),
      in_specs=[
          pl.BlockSpec((BLK, D), lambda i: (i, 0)),
          pl.BlockSpec((BLK, D), lambda i: (i + NB, 0)),
          pl.BlockSpec((BLK, DH), lambda i: (i, 0)),
          pl.BlockSpec((BLK, DH), lambda i: (i + NB, 0)),
      ],
      out_specs=pl.BlockSpec((BLK, DH), lambda i: (i, 0)),
      out_shape=jax.ShapeDtypeStruct((N, DH), jnp.float32),
  )(num, num, den, den)


def _prep_weights(W, a):
  wf = W.transpose(1, 0, 2).reshape(W.shape[1], D)
  ws = jnp.einsum('hdk,hk->dh', W, a[:, :DH])
  wd = jnp.einsum('hdk,hk->dh', W, a[:, DH:])
  pad = jnp.zeros((W.shape[1], DH - H), jnp.float32)
  return wf, jnp.concatenate([ws, pad], 1), jnp.concatenate([wd, pad], 1)


def kernel(x, adj, W1, a1, W2, a2):
  src = adj[0]
  dst = adj[1]
  w1f, ws1, wd1 = _prep_weights(W1, a1)
  w2f, ws2, wd2 = _prep_weights(W2, a2)

  h1, fs1, fd1 = _proj(x, w1f, ws1, wd1)
  num1, den1 = _edge_pass(src, dst, h1, fs1, fd1)
  h2, fs2, fd2 = _mid(num1, den1, w2f, ws2, wd2)
  num2, den2 = _edge_pass(src, dst, h2, fs2, fd2)
  return _final(num2, den2)


# speedup vs baseline: 156.9689x; 1.2180x over previous
import functools

import jax
import jax.numpy as jnp
from jax import lax
from jax.experimental import pallas as pl
from jax.experimental.pallas import tpu as pltpu
from jax.experimental.pallas import tpu_sc as plsc

N = 10000
E = 320000
D = 128
H = 8
DH = 16
ALPHA = 0.2

NC = 2
NS = 16
CH = 128
NCHUNK = E // CH
CHUNK_PER_CORE = NCHUNK // NC
ROWS_PER_TILE = N // NS
ZROWS = 125

BLK = 1000
NB = N // BLK


def _edge_body(idx_hbm, h_hbm, fs_hbm, fd_hbm,
               num_out, den_out,
               idxq0, idxq1, bufh0, bufh1, bufs, bufd, bufatt,
               accn, accd, sem_h, sem_f, sem_sc):
  c = lax.axis_index("c")
  s = lax.axis_index("s")

  def zero_loop(i, _):
    for q in range(D // 16):
      bufh0[i, pl.ds(q * 16, 16)] = jnp.zeros((16,), jnp.float32)
    bufs[i] = jnp.zeros((16,), jnp.float32)
    return 0
  lax.fori_loop(0, CH, zero_loop, 0)

  row0 = s * ROWS_PER_TILE
  for p in range(ROWS_PER_TILE // ZROWS):
    pltpu.sync_copy(bufh0.at[pl.ds(0, ZROWS)],
                    accn.at[pl.ds(row0 + p * ZROWS, ZROWS)])
    pltpu.sync_copy(bufs.at[pl.ds(0, ZROWS)],
                    accd.at[pl.ds(row0 + p * ZROWS, ZROWS)])
  plsc.subcore_barrier()

  def fetch_pair(p, qb):
    pltpu.sync_copy(idx_hbm.at[pl.ds(p, 2)], qb)

  def fire_h(qb, k, bh):
    pltpu.async_copy(h_hbm.at[qb.at[k, 0]], bh, sem_h)

  def fire_f(qb, k):
    pltpu.async_copy(fs_hbm.at[qb.at[k, 0]], bufs, sem_f)
    pltpu.async_copy(fd_hbm.at[qb.at[k, 1]], bufd, sem_f)

  def wait_h(qb, k, bh):
    pltpu.make_async_copy(h_hbm.at[qb.at[k, 0]], bh, sem_h).wait()

  def wait_f(qb, k):
    pltpu.make_async_copy(fs_hbm.at[qb.at[k, 0]], bufs, sem_f).wait()
    pltpu.make_async_copy(fd_hbm.at[qb.at[k, 1]], bufd, sem_f).wait()

  def att_phase():
    @plsc.parallel_loop(0, CH, unroll=8)
    def _(j):
      v = bufs[j] + bufd[j]
      bufatt[j] = jnp.exp(jnp.where(v >= 0.0, v, v * ALPHA))

  def scale_phase(bh):
    @plsc.parallel_loop(0, CH, unroll=4)
    def _(j):
      jrow = jnp.full((16,), j, jnp.int32)
      for h in range(H):
        b = plsc.load_gather(bufatt, [jrow, jnp.full((16,), h, jnp.int32)])
        bh[j, pl.ds(h * DH, DH)] = bh[j, pl.ds(h * DH, DH)] * b

  def scatter_async(bh, qb, k):
    pltpu.async_copy(bh, accn.at[qb.at[k, 1]], sem_sc, add=True)
    pltpu.sync_copy(bufatt, accd.at[qb.at[k, 1]], add=True)

  def wait_sc(bh, qb, k):
    pltpu.make_async_copy(bh, accn.at[qb.at[k, 1]], sem_sc).wait()

  base = c * CHUNK_PER_CORE + s * 78

  def do_pair(p, tq, xq, wait_prev, fire_next):
    wait_f(tq, 0)
    att_phase()
    if wait_prev:
      wait_sc(bufh1, xq, 1)
    fetch_pair(p + 2, xq)
    wait_h(tq, 0, bufh0)
    fire_f(tq, 1)
    fire_h(tq, 1, bufh1)
    scale_phase(bufh0)
    scatter_async(bufh0, tq, 0)

    wait_f(tq, 1)
    att_phase()
    wait_sc(bufh0, tq, 0)
    wait_h(tq, 1, bufh1)
    if fire_next:
      fire_f(xq, 0)
      fire_h(xq, 0, bufh0)
    scale_phase(bufh1)
    scatter_async(bufh1, tq, 1)

  fetch_pair(base, idxq0)
  fire_f(idxq0, 0)
  fire_h(idxq0, 0, bufh0)

  def quad(u, _):
    @pl.when(u > 0)
    def _():
      wait_sc(bufh1, idxq1, 1)

    pA = base + 4 * u
    wait_f(idxq0, 0)
    att_phase()
    fetch_pair(pA + 2, idxq1)
    wait_h(idxq0, 0, bufh0)
    fire_f(idxq0, 1)
    fire_h(idxq0, 1, bufh1)
    scale_phase(bufh0)
    scatter_async(bufh0, idxq0, 0)

    wait_f(idxq0, 1)
    att_phase()
    wait_sc(bufh0, idxq0, 0)
    wait_h(idxq0, 1, bufh1)
    fire_f(idxq1, 0)
    fire_h(idxq1, 0, bufh0)
    scale_phase(bufh1)
    scatter_async(bufh1, idxq0, 1)

    wait_f(idxq1, 0)
    att_phase()
    wait_sc(bufh1, idxq0, 1)
    fetch_pair(pA + 4, idxq0)
    wait_h(idxq1, 0, bufh0)
    fire_f(idxq1, 1)
    fire_h(idxq1, 1, bufh1)
    scale_phase(bufh0)
    scatter_async(bufh0, idxq1, 0)

    wait_f(idxq1, 1)
    att_phase()
    wait_sc(bufh0, idxq1, 0)
    wait_h(idxq1, 1, bufh1)
    fire_f(idxq0, 0)
    fire_h(idxq0, 0, bufh0)
    scale_phase(bufh1)
    scatter_async(bufh1, idxq1, 1)
    return 0
  lax.fori_loop(0, 76 // 4, quad, 0)

  do_pair(base + 76, idxq0, idxq1, wait_prev=True, fire_next=False)
  wait_sc(bufh1, idxq0, 1)

  @pl.when(s < CHUNK_PER_CORE - NS * 78)
  def _():
    cid = c * CHUNK_PER_CORE + NS * 78 + s
    pltpu.sync_copy(idx_hbm.at[pl.ds(cid, 1)], idxq0.at[pl.ds(0, 1)])
    fire_f(idxq0, 0)
    fire_h(idxq0, 0, bufh0)
    wait_f(idxq0, 0)
    wait_h(idxq0, 0, bufh0)
    att_phase()
    scale_phase(bufh0)
    pltpu.sync_copy(bufh0, accn.at[idxq0.at[0, 1]], add=True)
    pltpu.sync_copy(bufatt, accd.at[idxq0.at[0, 1]], add=True)

  plsc.subcore_barrier()

  out_row0 = c * N + s * ROWS_PER_TILE
  for p in range(ROWS_PER_TILE // ZROWS):
    pltpu.sync_copy(accn.at[pl.ds(row0 + p * ZROWS, ZROWS)],
                    num_out.at[pl.ds(out_row0 + p * ZROWS, ZROWS)])
    pltpu.sync_copy(accd.at[pl.ds(row0 + p * ZROWS, ZROWS)],
                    den_out.at[pl.ds(out_row0 + p * ZROWS, ZROWS)])


_edge_pass = functools.partial(
    pl.kernel,
    out_type=(
        jax.ShapeDtypeStruct((NC * N, D), jnp.float32),
        jax.ShapeDtypeStruct((NC * N, DH), jnp.float32),
    ),
    mesh=plsc.VectorSubcoreMesh(core_axis_name="c", subcore_axis_name="s"),
    scratch_types=[
        pltpu.VMEM((2, 2, CH), jnp.int32),
        pltpu.VMEM((2, 2, CH), jnp.int32),
        pltpu.VMEM((CH, D), jnp.float32),
        pltpu.VMEM((CH, D), jnp.float32),
        pltpu.VMEM((CH, DH), jnp.float32),
        pltpu.VMEM((CH, DH), jnp.float32),
        pltpu.VMEM((CH, DH), jnp.float32),
        pltpu.VMEM_SHARED((N, D), jnp.float32),
        pltpu.VMEM_SHARED((N, DH), jnp.float32),
        pltpu.SemaphoreType.DMA,
        pltpu.SemaphoreType.DMA,
        pltpu.SemaphoreType.DMA,
    ],
    compiler_params=pltpu.CompilerParams(
        use_tc_tiling_on_sc=False, needs_layout_passes=False),
)(_edge_body)


def _proj_body(x_ref, w_ref, ws_ref, wd_ref, h_ref, fs_ref, fd_ref):
  xb = x_ref[...]
  h_ref[...] = jnp.dot(xb, w_ref[...], preferred_element_type=jnp.float32)
  fs_ref[...] = jnp.dot(xb, ws_ref[...], preferred_element_type=jnp.float32)
  fd_ref[...] = jnp.dot(xb, wd_ref[...], preferred_element_type=jnp.float32)


def _proj(x, w, ws, wd):
  return pl.pallas_call(
      _proj_body,
      grid=(NB,),
      in_specs=[
          pl.BlockSpec((BLK, D), lambda i: (i, 0)),
          pl.BlockSpec((D, D), lambda i: (0, 0)),
          pl.BlockSpec((D, DH), lambda i: (0, 0)),
          pl.BlockSpec((D, DH), lambda i: (0, 0)),
      ],
      out_specs=[
          pl.BlockSpec((BLK, D), lambda i: (i, 0)),
          pl.BlockSpec((BLK, DH), lambda i: (i, 0)),
          pl.BlockSpec((BLK, DH), lambda i: (i, 0)),
      ],
      out_shape=[
          jax.ShapeDtypeStruct((N, D), jnp.float32),
          jax.ShapeDtypeStruct((N, DH), jnp.float32),
          jax.ShapeDtypeStruct((N, DH), jnp.float32),
      ],
  )(x, w, ws, wd)


def _head_expand():
  row = lax.broadcasted_iota(jnp.int32, (DH, D), 0)
  col = lax.broadcasted_iota(jnp.int32, (DH, D), 1)
  return (row == col // DH).astype(jnp.float32)


def _normalize(n0, n1, d0, d1):
  num = n0 + n1
  den = jnp.dot(d0 + d1, _head_expand(), preferred_element_type=jnp.float32)
  return num / (den + 1e-16)


def _mid_body(n0_ref, n1_ref, d0_ref, d1_ref, w_ref, ws_ref, wd_ref,
              h_ref, fs_ref, fd_ref):
  r = _normalize(n0_ref[...], n1_ref[...], d0_ref[...], d1_ref[...])
  hcat = jnp.where(r > 0.0, r, jnp.exp(jnp.minimum(r, 0.0)) - 1.0)
  h_ref[...] = jnp.dot(hcat, w_ref[...], preferred_element_type=jnp.float32)
  fs_ref[...] = jnp.dot(hcat, ws_ref[...], preferred_element_type=jnp.float32)
  fd_ref[...] = jnp.dot(hcat, wd_ref[...], preferred_element_type=jnp.float32)


def _mid(num, den, w, ws, wd):
  return pl.pallas_call(
      _mid_body,
      grid=(NB,),
      in_specs=[
          pl.BlockSpec((BLK, D), lambda i: (i, 0)),
          pl.BlockSpec((BLK, D), lambda i: (i + NB, 0)),
          pl.BlockSpec((BLK, DH), lambda i: (i, 0)),
          pl.BlockSpec((BLK, DH), lambda i: (i + NB, 0)),
          pl.BlockSpec((D, D), lambda i: (0, 0)),
          pl.BlockSpec((D, DH), lambda i: (0, 0)),
          pl.BlockSpec((D, DH), lambda i: (0, 0)),
      ],
      out_specs=[
          pl.BlockSpec((BLK, D), lambda i: (i, 0)),
          pl.BlockSpec((BLK, DH), lambda i: (i, 0)),
          pl.BlockSpec((BLK, DH), lambda i: (i, 0)),
      ],
      out_shape=[
          jax.ShapeDtypeStruct((N, D), jnp.float32),
          jax.ShapeDtypeStruct((N, DH), jnp.float32),
          jax.ShapeDtypeStruct((N, DH), jnp.float32),
      ],
  )(num, num, den, den, w, ws, wd)


def _final_body(n0_ref, n1_ref, d0_ref, d1_ref, o_ref):
  r = _normalize(n0_ref[...], n1_ref[...], d0_ref[...], d1_ref[...])
  row = lax.broadcasted_iota(jnp.int32, (D, DH), 0)
  col = lax.broadcasted_iota(jnp.int32, (D, DH), 1)
  t = (row % DH == col).astype(jnp.float32) * (1.0 / H)
  o_ref[...] = jnp.dot(r, t, preferred_element_type=jnp.float32)


def _final(num, den):
  return pl.pallas_call(
      _final_body,
      grid=(NB,),
      in_specs=[
          pl.BlockSpec((BLK, D), lambda i: (i, 0)),
          pl.BlockSpec((BLK, D), lambda i: (i + NB, 0)),
          pl.BlockSpec((BLK, DH), lambda i: (i, 0)),
          pl.BlockSpec((BLK, DH), lambda i: (i + NB, 0)),
      ],
      out_specs=pl.BlockSpec((BLK, DH), lambda i: (i, 0)),
      out_shape=jax.ShapeDtypeStruct((N, DH), jnp.float32),
  )(num, num, den, den)


def _prep_weights(W, a):
  wf = W.transpose(1, 0, 2).reshape(W.shape[1], D)
  ws = jnp.einsum('hdk,hk->dh', W, a[:, :DH])
  wd = jnp.einsum('hdk,hk->dh', W, a[:, DH:])
  pad = jnp.zeros((W.shape[1], DH - H), jnp.float32)
  return wf, jnp.concatenate([ws, pad], 1), jnp.concatenate([wd, pad], 1)


def kernel(x, adj, W1, a1, W2, a2):
  idx = jnp.stack([adj[0].reshape(NCHUNK, CH), adj[1].reshape(NCHUNK, CH)],
                  axis=1)
  w1f, ws1, wd1 = _prep_weights(W1, a1)
  w2f, ws2, wd2 = _prep_weights(W2, a2)

  h1, fs1, fd1 = _proj(x, w1f, ws1, wd1)
  num1, den1 = _edge_pass(idx, h1, fs1, fd1)
  h2, fs2, fd2 = _mid(num1, den1, w2f, ws2, wd2)
  num2, den2 = _edge_pass(idx, h2, fs2, fd2)
  return _final(num2, den2)


# speedup vs baseline: 178.9467x; 1.1400x over previous
import functools

import jax
import jax.numpy as jnp
from jax import lax
from jax.experimental import pallas as pl
from jax.experimental.pallas import tpu as pltpu
from jax.experimental.pallas import tpu_sc as plsc

N = 10000
E = 320000
D = 128
H = 8
DH = 16
ALPHA = 0.2

NC = 2
NS = 16
CH = 128
NCHUNK = E // CH
CHUNK_PER_CORE = NCHUNK // NC
ROWS_PER_TILE = N // NS
ZROWS = 125

BLK = 1000
NB = N // BLK


def _edge_body(idx_hbm, h_hbm, fs_hbm, fd_hbm,
               num_out, den_out,
               idxq0, idxq1, bufh0, bufh1, bufs, bufd, bufatt,
               accn, accd, sem_h, sem_f, sem_sc):
  c = lax.axis_index("c")
  s = lax.axis_index("s")

  def zero_loop(i, _):
    for q in range(D // 16):
      bufh0[i, pl.ds(q * 16, 16)] = jnp.zeros((16,), jnp.float32)
    bufs[i] = jnp.zeros((16,), jnp.float32)
    return 0
  lax.fori_loop(0, CH, zero_loop, 0)

  row0 = s * ROWS_PER_TILE
  for p in range(ROWS_PER_TILE // ZROWS):
    pltpu.sync_copy(bufh0.at[pl.ds(0, ZROWS)],
                    accn.at[pl.ds(row0 + p * ZROWS, ZROWS)])
    pltpu.sync_copy(bufs.at[pl.ds(0, ZROWS)],
                    accd.at[pl.ds(row0 + p * ZROWS, ZROWS)])
  plsc.subcore_barrier()

  def fetch_pair(p, qb):
    pltpu.sync_copy(idx_hbm.at[pl.ds(p, 2)], qb)

  def fire_h(qb, k, bh):
    pltpu.async_copy(h_hbm.at[qb.at[k, 0]], bh, sem_h)

  def fire_f(qb, k):
    pltpu.async_copy(fs_hbm.at[qb.at[k, 0]], bufs, sem_f)
    pltpu.async_copy(fd_hbm.at[qb.at[k, 1]], bufd, sem_f)

  def wait_h(qb, k, bh):
    pltpu.make_async_copy(h_hbm.at[qb.at[k, 0]], bh, sem_h).wait()

  def wait_f(qb, k):
    pltpu.make_async_copy(fs_hbm.at[qb.at[k, 0]], bufs, sem_f).wait()
    pltpu.make_async_copy(fd_hbm.at[qb.at[k, 1]], bufd, sem_f).wait()

  def att_phase():
    @plsc.parallel_loop(0, CH, unroll=8)
    def _(j):
      v = bufs[j] + bufd[j]
      bufatt[j] = jnp.exp(jnp.where(v >= 0.0, v, v * ALPHA))

  def scale_phase(bh):
    @plsc.parallel_loop(0, CH, unroll=4)
    def _(j):
      arow = bufatt[j]
      for h in range(H):
        b = jnp.broadcast_to(arow[h], (16,))
        bh[j, pl.ds(h * DH, DH)] = bh[j, pl.ds(h * DH, DH)] * b

  def scatter_async(bh, qb, k):
    pltpu.async_copy(bh, accn.at[qb.at[k, 1]], sem_sc, add=True)
    pltpu.sync_copy(bufatt, accd.at[qb.at[k, 1]], add=True)

  def wait_sc(bh, qb, k):
    pltpu.make_async_copy(bh, accn.at[qb.at[k, 1]], sem_sc).wait()

  base = c * CHUNK_PER_CORE + s * 78

  def do_pair(p, tq, xq, wait_prev, fire_next):
    wait_f(tq, 0)
    att_phase()
    if wait_prev:
      wait_sc(bufh1, xq, 1)
    fetch_pair(p + 2, xq)
    wait_h(tq, 0, bufh0)
    fire_f(tq, 1)
    fire_h(tq, 1, bufh1)
    scale_phase(bufh0)
    scatter_async(bufh0, tq, 0)

    wait_f(tq, 1)
    att_phase()
    wait_sc(bufh0, tq, 0)
    wait_h(tq, 1, bufh1)
    if fire_next:
      fire_f(xq, 0)
      fire_h(xq, 0, bufh0)
    scale_phase(bufh1)
    scatter_async(bufh1, tq, 1)

  fetch_pair(base, idxq0)
  fire_f(idxq0, 0)
  fire_h(idxq0, 0, bufh0)

  def quad(u, _):
    @pl.when(u > 0)
    def _():
      wait_sc(bufh1, idxq1, 1)

    pA = base + 4 * u
    wait_f(idxq0, 0)
    att_phase()
    fetch_pair(pA + 2, idxq1)
    wait_h(idxq0, 0, bufh0)
    fire_f(idxq0, 1)
    fire_h(idxq0, 1, bufh1)
    scale_phase(bufh0)
    scatter_async(bufh0, idxq0, 0)

    wait_f(idxq0, 1)
    att_phase()
    wait_sc(bufh0, idxq0, 0)
    wait_h(idxq0, 1, bufh1)
    fire_f(idxq1, 0)
    fire_h(idxq1, 0, bufh0)
    scale_phase(bufh1)
    scatter_async(bufh1, idxq0, 1)

    wait_f(idxq1, 0)
    att_phase()
    wait_sc(bufh1, idxq0, 1)
    fetch_pair(pA + 4, idxq0)
    wait_h(idxq1, 0, bufh0)
    fire_f(idxq1, 1)
    fire_h(idxq1, 1, bufh1)
    scale_phase(bufh0)
    scatter_async(bufh0, idxq1, 0)

    wait_f(idxq1, 1)
    att_phase()
    wait_sc(bufh0, idxq1, 0)
    wait_h(idxq1, 1, bufh1)
    fire_f(idxq0, 0)
    fire_h(idxq0, 0, bufh0)
    scale_phase(bufh1)
    scatter_async(bufh1, idxq1, 1)
    return 0
  lax.fori_loop(0, 76 // 4, quad, 0)

  do_pair(base + 76, idxq0, idxq1, wait_prev=True, fire_next=False)
  wait_sc(bufh1, idxq0, 1)

  @pl.when(s < CHUNK_PER_CORE - NS * 78)
  def _():
    cid = c * CHUNK_PER_CORE + NS * 78 + s
    pltpu.sync_copy(idx_hbm.at[pl.ds(cid, 1)], idxq0.at[pl.ds(0, 1)])
    fire_f(idxq0, 0)
    fire_h(idxq0, 0, bufh0)
    wait_f(idxq0, 0)
    wait_h(idxq0, 0, bufh0)
    att_phase()
    scale_phase(bufh0)
    pltpu.sync_copy(bufh0, accn.at[idxq0.at[0, 1]], add=True)
    pltpu.sync_copy(bufatt, accd.at[idxq0.at[0, 1]], add=True)

  plsc.subcore_barrier()

  out_row0 = c * N + s * ROWS_PER_TILE
  for p in range(ROWS_PER_TILE // ZROWS):
    pltpu.sync_copy(accn.at[pl.ds(row0 + p * ZROWS, ZROWS)],
                    num_out.at[pl.ds(out_row0 + p * ZROWS, ZROWS)])
    pltpu.sync_copy(accd.at[pl.ds(row0 + p * ZROWS, ZROWS)],
                    den_out.at[pl.ds(out_row0 + p * ZROWS, ZROWS)])


_edge_pass = functools.partial(
    pl.kernel,
    out_type=(
        jax.ShapeDtypeStruct((NC * N, D), jnp.float32),
        jax.ShapeDtypeStruct((NC * N, DH), jnp.float32),
    ),
    mesh=plsc.VectorSubcoreMesh(core_axis_name="c", subcore_axis_name="s"),
    scratch_types=[
        pltpu.VMEM((2, 2, CH), jnp.int32),
        pltpu.VMEM((2, 2, CH), jnp.int32),
        pltpu.VMEM((CH, D), jnp.float32),
        pltpu.VMEM((CH, D), jnp.float32),
        pltpu.VMEM((CH, DH), jnp.float32),
        pltpu.VMEM((CH, DH), jnp.float32),
        pltpu.VMEM((CH, DH), jnp.float32),
        pltpu.VMEM_SHARED((N, D), jnp.float32),
        pltpu.VMEM_SHARED((N, DH), jnp.float32),
        pltpu.SemaphoreType.DMA,
        pltpu.SemaphoreType.DMA,
        pltpu.SemaphoreType.DMA,
    ],
    compiler_params=pltpu.CompilerParams(
        use_tc_tiling_on_sc=False, needs_layout_passes=False),
)(_edge_body)


def _proj_body(x_ref, w_ref, ws_ref, wd_ref, h_ref, fs_ref, fd_ref):
  xb = x_ref[...]
  h_ref[...] = jnp.dot(xb, w_ref[...], preferred_element_type=jnp.float32)
  fs_ref[...] = jnp.dot(xb, ws_ref[...], preferred_element_type=jnp.float32)
  fd_ref[...] = jnp.dot(xb, wd_ref[...], preferred_element_type=jnp.float32)


def _proj(x, w, ws, wd):
  return pl.pallas_call(
      _proj_body,
      grid=(NB,),
      in_specs=[
          pl.BlockSpec((BLK, D), lambda i: (i, 0)),
          pl.BlockSpec((D, D), lambda i: (0, 0)),
          pl.BlockSpec((D, DH), lambda i: (0, 0)),
          pl.BlockSpec((D, DH), lambda i: (0, 0)),
      ],
      out_specs=[
          pl.BlockSpec((BLK, D), lambda i: (i, 0)),
          pl.BlockSpec((BLK, DH), lambda i: (i, 0)),
          pl.BlockSpec((BLK, DH), lambda i: (i, 0)),
      ],
      out_shape=[
          jax.ShapeDtypeStruct((N, D), jnp.float32),
          jax.ShapeDtypeStruct((N, DH), jnp.float32),
          jax.ShapeDtypeStruct((N, DH), jnp.float32),
      ],
  )(x, w, ws, wd)


def _head_expand():
  row = lax.broadcasted_iota(jnp.int32, (DH, D), 0)
  col = lax.broadcasted_iota(jnp.int32, (DH, D), 1)
  return (row == col // DH).astype(jnp.float32)


def _normalize(n0, n1, d0, d1):
  num = n0 + n1
  den = jnp.dot(d0 + d1, _head_expand(), preferred_element_type=jnp.float32)
  return num / (den + 1e-16)


def _mid_body(n0_ref, n1_ref, d0_ref, d1_ref, w_ref, ws_ref, wd_ref,
              h_ref, fs_ref, fd_ref):
  r = _normalize(n0_ref[...], n1_ref[...], d0_ref[...], d1_ref[...])
  hcat = jnp.where(r > 0.0, r, jnp.exp(jnp.minimum(r, 0.0)) - 1.0)
  h_ref[...] = jnp.dot(hcat, w_ref[...], preferred_element_type=jnp.float32)
  fs_ref[...] = jnp.dot(hcat, ws_ref[...], preferred_element_type=jnp.float32)
  fd_ref[...] = jnp.dot(hcat, wd_ref[...], preferred_element_type=jnp.float32)


def _mid(num, den, w, ws, wd):
  return pl.pallas_call(
      _mid_body,
      grid=(NB,),
      in_specs=[
          pl.BlockSpec((BLK, D), lambda i: (i, 0)),
          pl.BlockSpec((BLK, D), lambda i: (i + NB, 0)),
          pl.BlockSpec((BLK, DH), lambda i: (i, 0)),
          pl.BlockSpec((BLK, DH), lambda i: (i + NB, 0)),
          pl.BlockSpec((D, D), lambda i: (0, 0)),
          pl.BlockSpec((D, DH), lambda i: (0, 0)),
          pl.BlockSpec((D, DH), lambda i: (0, 0)),
      ],
      out_specs=[
          pl.BlockSpec((BLK, D), lambda i: (i, 0)),
          pl.BlockSpec((BLK, DH), lambda i: (i, 0)),
          pl.BlockSpec((BLK, DH), lambda i: (i, 0)),
      ],
      out_shape=[
          jax.ShapeDtypeStruct((N, D), jnp.float32),
          jax.ShapeDtypeStruct((N, DH), jnp.float32),
          jax.ShapeDtypeStruct((N, DH), jnp.float32),
      ],
  )(num, num, den, den, w, ws, wd)


def _final_body(n0_ref, n1_ref, d0_ref, d1_ref, o_ref):
  r = _normalize(n0_ref[...], n1_ref[...], d0_ref[...], d1_ref[...])
  row = lax.broadcasted_iota(jnp.int32, (D, DH), 0)
  col = lax.broadcasted_iota(jnp.int32, (D, DH), 1)
  t = (row % DH == col).astype(jnp.float32) * (1.0 / H)
  o_ref[...] = jnp.dot(r, t, preferred_element_type=jnp.float32)


def _final(num, den):
  return pl.pallas_call(
      _final_body,
      grid=(NB,),
      in_specs=[
          pl.BlockSpec((BLK, D), lambda i: (i, 0)),
          pl.BlockSpec((BLK, D), lambda i: (i + NB, 0)),
          pl.BlockSpec((BLK, DH), lambda i: (i, 0)),
          pl.BlockSpec((BLK, DH), lambda i: (i + NB, 0)),
      ],
      out_specs=pl.BlockSpec((BLK, DH), lambda i: (i, 0)),
      out_shape=jax.ShapeDtypeStruct((N, DH), jnp.float32),
  )(num, num, den, den)


def _prep_weights(W, a):
  wf = W.transpose(1, 0, 2).reshape(W.shape[1], D)
  ws = jnp.einsum('hdk,hk->dh', W, a[:, :DH])
  wd = jnp.einsum('hdk,hk->dh', W, a[:, DH:])
  pad = jnp.zeros((W.shape[1], DH - H), jnp.float32)
  return wf, jnp.concatenate([ws, pad], 1), jnp.concatenate([wd, pad], 1)


def kernel(x, adj, W1, a1, W2, a2):
  idx = jnp.stack([adj[0].reshape(NCHUNK, CH), adj[1].reshape(NCHUNK, CH)],
                  axis=1)
  w1f, ws1, wd1 = _prep_weights(W1, a1)
  w2f, ws2, wd2 = _prep_weights(W2, a2)

  h1, fs1, fd1 = _proj(x, w1f, ws1, wd1)
  num1, den1 = _edge_pass(idx, h1, fs1, fd1)
  h2, fs2, fd2 = _mid(num1, den1, w2f, ws2, wd2)
  num2, den2 = _edge_pass(idx, h2, fs2, fd2)
  return _final(num2, den2)


# speedup vs baseline: 179.2006x; 1.0014x over previous
import functools

import jax
import jax.numpy as jnp
from jax import lax
from jax.experimental import pallas as pl
from jax.experimental.pallas import tpu as pltpu
from jax.experimental.pallas import tpu_sc as plsc

N = 10000
E = 320000
D = 128
H = 8
DH = 16
ALPHA = 0.2

NC = 2
NS = 16
CH = 128
NCHUNK = E // CH
CHUNK_PER_CORE = NCHUNK // NC
ROWS_PER_TILE = N // NS
ZROWS = 125

BLK = 1000
NB = N // BLK


def _edge_body(idx_hbm, h_hbm, fs_hbm, fd_hbm,
               num_out, den_out,
               idxq0, idxq1, bufh0, bufh1, bufs, bufd, bufatt,
               accn, accd, sem_h, sem_f, sem_sc, sem_sa):
  c = lax.axis_index("c")
  s = lax.axis_index("s")

  def zero_loop(i, _):
    for q in range(D // 16):
      bufh0[i, pl.ds(q * 16, 16)] = jnp.zeros((16,), jnp.float32)
    bufs[i] = jnp.zeros((16,), jnp.float32)
    return 0
  lax.fori_loop(0, CH, zero_loop, 0)

  row0 = s * ROWS_PER_TILE
  for p in range(ROWS_PER_TILE // ZROWS):
    pltpu.sync_copy(bufh0.at[pl.ds(0, ZROWS)],
                    accn.at[pl.ds(row0 + p * ZROWS, ZROWS)])
    pltpu.sync_copy(bufs.at[pl.ds(0, ZROWS)],
                    accd.at[pl.ds(row0 + p * ZROWS, ZROWS)])
  plsc.subcore_barrier()

  def fetch_pair(p, qb):
    pltpu.sync_copy(idx_hbm.at[pl.ds(p, 2)], qb)

  def fire_h(qb, k, bh):
    pltpu.async_copy(h_hbm.at[qb.at[k, 0]], bh, sem_h)

  def fire_f(qb, k):
    pltpu.async_copy(fs_hbm.at[qb.at[k, 0]], bufs, sem_f)
    pltpu.async_copy(fd_hbm.at[qb.at[k, 1]], bufd, sem_f)

  def wait_h(qb, k, bh):
    pltpu.make_async_copy(h_hbm.at[qb.at[k, 0]], bh, sem_h).wait()

  def wait_f(qb, k):
    pltpu.make_async_copy(fs_hbm.at[qb.at[k, 0]], bufs, sem_f).wait()
    pltpu.make_async_copy(fd_hbm.at[qb.at[k, 1]], bufd, sem_f).wait()

  def att_phase():
    @plsc.parallel_loop(0, CH, unroll=8)
    def _(j):
      v = bufs[j] + bufd[j]
      bufatt[j] = jnp.exp(jnp.where(v >= 0.0, v, v * ALPHA))

  def scale_phase(bh):
    @plsc.parallel_loop(0, CH, unroll=4)
    def _(j):
      arow = bufatt[j]
      for h in range(H):
        b = jnp.broadcast_to(arow[h], (16,))
        bh[j, pl.ds(h * DH, DH)] = bh[j, pl.ds(h * DH, DH)] * b

  def fire_a(qb, k):
    pltpu.async_copy(bufatt, accd.at[qb.at[k, 1]], sem_sa, add=True)

  def wait_a(qb, k):
    pltpu.make_async_copy(bufatt, accd.at[qb.at[k, 1]], sem_sa).wait()

  def scatter_async(bh, qb, k):
    pltpu.async_copy(bh, accn.at[qb.at[k, 1]], sem_sc, add=True)

  def wait_sc(bh, qb, k):
    pltpu.make_async_copy(bh, accn.at[qb.at[k, 1]], sem_sc).wait()

  base = c * CHUNK_PER_CORE + s * 78

  def do_pair(p, tq, xq, wait_prev, fire_next):
    wait_f(tq, 0)
    if wait_prev:
      wait_a(xq, 1)
    att_phase()
    fire_a(tq, 0)
    if wait_prev:
      wait_sc(bufh1, xq, 1)
    fetch_pair(p + 2, xq)
    wait_h(tq, 0, bufh0)
    fire_f(tq, 1)
    fire_h(tq, 1, bufh1)
    scale_phase(bufh0)
    scatter_async(bufh0, tq, 0)

    wait_f(tq, 1)
    wait_a(tq, 0)
    att_phase()
    fire_a(tq, 1)
    wait_sc(bufh0, tq, 0)
    wait_h(tq, 1, bufh1)
    if fire_next:
      fire_f(xq, 0)
      fire_h(xq, 0, bufh0)
    scale_phase(bufh1)
    scatter_async(bufh1, tq, 1)

  fetch_pair(base, idxq0)
  fire_f(idxq0, 0)
  fire_h(idxq0, 0, bufh0)

  def quad(u, _):
    @pl.when(u > 0)
    def _():
      wait_a(idxq1, 1)
    @pl.when(u > 0)
    def _():
      wait_sc(bufh1, idxq1, 1)

    pA = base + 4 * u
    wait_f(idxq0, 0)
    att_phase()
    fire_a(idxq0, 0)
    fetch_pair(pA + 2, idxq1)
    wait_h(idxq0, 0, bufh0)
    fire_f(idxq0, 1)
    fire_h(idxq0, 1, bufh1)
    scale_phase(bufh0)
    scatter_async(bufh0, idxq0, 0)

    wait_f(idxq0, 1)
    wait_a(idxq0, 0)
    att_phase()
    fire_a(idxq0, 1)
    wait_sc(bufh0, idxq0, 0)
    wait_h(idxq0, 1, bufh1)
    fire_f(idxq1, 0)
    fire_h(idxq1, 0, bufh0)
    scale_phase(bufh1)
    scatter_async(bufh1, idxq0, 1)

    wait_f(idxq1, 0)
    wait_a(idxq0, 1)
    att_phase()
    fire_a(idxq1, 0)
    wait_sc(bufh1, idxq0, 1)
    fetch_pair(pA + 4, idxq0)
    wait_h(idxq1, 0, bufh0)
    fire_f(idxq1, 1)
    fire_h(idxq1, 1, bufh1)
    scale_phase(bufh0)
    scatter_async(bufh0, idxq1, 0)

    wait_f(idxq1, 1)
    wait_a(idxq1, 0)
    att_phase()
    fire_a(idxq1, 1)
    wait_sc(bufh0, idxq1, 0)
    wait_h(idxq1, 1, bufh1)
    fire_f(idxq0, 0)
    fire_h(idxq0, 0, bufh0)
    scale_phase(bufh1)
    scatter_async(bufh1, idxq1, 1)
    return 0
  lax.fori_loop(0, 76 // 4, quad, 0)

  do_pair(base + 76, idxq0, idxq1, wait_prev=True, fire_next=False)
  wait_a(idxq0, 1)
  wait_sc(bufh1, idxq0, 1)

  @pl.when(s < CHUNK_PER_CORE - NS * 78)
  def _():
    cid = c * CHUNK_PER_CORE + NS * 78 + s
    pltpu.sync_copy(idx_hbm.at[pl.ds(cid, 1)], idxq0.at[pl.ds(0, 1)])
    fire_f(idxq0, 0)
    fire_h(idxq0, 0, bufh0)
    wait_f(idxq0, 0)
    wait_h(idxq0, 0, bufh0)
    att_phase()
    scale_phase(bufh0)
    pltpu.sync_copy(bufh0, accn.at[idxq0.at[0, 1]], add=True)
    pltpu.sync_copy(bufatt, accd.at[idxq0.at[0, 1]], add=True)

  plsc.subcore_barrier()

  out_row0 = c * N + s * ROWS_PER_TILE
  for p in range(ROWS_PER_TILE // ZROWS):
    pltpu.sync_copy(accn.at[pl.ds(row0 + p * ZROWS, ZROWS)],
                    num_out.at[pl.ds(out_row0 + p * ZROWS, ZROWS)])
    pltpu.sync_copy(accd.at[pl.ds(row0 + p * ZROWS, ZROWS)],
                    den_out.at[pl.ds(out_row0 + p * ZROWS, ZROWS)])


_edge_pass = functools.partial(
    pl.kernel,
    out_type=(
        jax.ShapeDtypeStruct((NC * N, D), jnp.float32),
        jax.ShapeDtypeStruct((NC * N, DH), jnp.float32),
    ),
    mesh=plsc.VectorSubcoreMesh(core_axis_name="c", subcore_axis_name="s"),
    scratch_types=[
        pltpu.VMEM((2, 2, CH), jnp.int32),
        pltpu.VMEM((2, 2, CH), jnp.int32),
        pltpu.VMEM((CH, D), jnp.float32),
        pltpu.VMEM((CH, D), jnp.float32),
        pltpu.VMEM((CH, DH), jnp.float32),
        pltpu.VMEM((CH, DH), jnp.float32),
        pltpu.VMEM((CH, DH), jnp.float32),
        pltpu.VMEM_SHARED((N, D), jnp.float32),
        pltpu.VMEM_SHARED((N, DH), jnp.float32),
        pltpu.SemaphoreType.DMA,
        pltpu.SemaphoreType.DMA,
        pltpu.SemaphoreType.DMA,
        pltpu.SemaphoreType.DMA,
    ],
    compiler_params=pltpu.CompilerParams(
        use_tc_tiling_on_sc=False, needs_layout_passes=False),
)(_edge_body)


def _proj_body(x_ref, w_ref, ws_ref, wd_ref, h_ref, fs_ref, fd_ref):
  xb = x_ref[...]
  h_ref[...] = jnp.dot(xb, w_ref[...], preferred_element_type=jnp.float32)
  fs_ref[...] = jnp.dot(xb, ws_ref[...], preferred_element_type=jnp.float32)
  fd_ref[...] = jnp.dot(xb, wd_ref[...], preferred_element_type=jnp.float32)


def _proj(x, w, ws, wd):
  return pl.pallas_call(
      _proj_body,
      grid=(NB,),
      in_specs=[
          pl.BlockSpec((BLK, D), lambda i: (i, 0)),
          pl.BlockSpec((D, D), lambda i: (0, 0)),
          pl.BlockSpec((D, DH), lambda i: (0, 0)),
          pl.BlockSpec((D, DH), lambda i: (0, 0)),
      ],
      out_specs=[
          pl.BlockSpec((BLK, D), lambda i: (i, 0)),
          pl.BlockSpec((BLK, DH), lambda i: (i, 0)),
          pl.BlockSpec((BLK, DH), lambda i: (i, 0)),
      ],
      out_shape=[
          jax.ShapeDtypeStruct((N, D), jnp.float32),
          jax.ShapeDtypeStruct((N, DH), jnp.float32),
          jax.ShapeDtypeStruct((N, DH), jnp.float32),
      ],
  )(x, w, ws, wd)


def _head_expand():
  row = lax.broadcasted_iota(jnp.int32, (DH, D), 0)
  col = lax.broadcasted_iota(jnp.int32, (DH, D), 1)
  return (row == col // DH).astype(jnp.float32)


def _normalize(n0, n1, d0, d1):
  num = n0 + n1
  den = jnp.dot(d0 + d1, _head_expand(), preferred_element_type=jnp.float32)
  return num / (den + 1e-16)


def _mid_body(n0_ref, n1_ref, d0_ref, d1_ref, w_ref, ws_ref, wd_ref,
              h_ref, fs_ref, fd_ref):
  r = _normalize(n0_ref[...], n1_ref[...], d0_ref[...], d1_ref[...])
  hcat = jnp.where(r > 0.0, r, jnp.exp(jnp.minimum(r, 0.0)) - 1.0)
  h_ref[...] = jnp.dot(hcat, w_ref[...], preferred_element_type=jnp.float32)
  fs_ref[...] = jnp.dot(hcat, ws_ref[...], preferred_element_type=jnp.float32)
  fd_ref[...] = jnp.dot(hcat, wd_ref[...], preferred_element_type=jnp.float32)


def _mid(num, den, w, ws, wd):
  return pl.pallas_call(
      _mid_body,
      grid=(NB,),
      in_specs=[
          pl.BlockSpec((BLK, D), lambda i: (i, 0)),
          pl.BlockSpec((BLK, D), lambda i: (i + NB, 0)),
          pl.BlockSpec((BLK, DH), lambda i: (i, 0)),
          pl.BlockSpec((BLK, DH), lambda i: (i + NB, 0)),
          pl.BlockSpec((D, D), lambda i: (0, 0)),
          pl.BlockSpec((D, DH), lambda i: (0, 0)),
          pl.BlockSpec((D, DH), lambda i: (0, 0)),
      ],
      out_specs=[
          pl.BlockSpec((BLK, D), lambda i: (i, 0)),
          pl.BlockSpec((BLK, DH), lambda i: (i, 0)),
          pl.BlockSpec((BLK, DH), lambda i: (i, 0)),
      ],
      out_shape=[
          jax.ShapeDtypeStruct((N, D), jnp.float32),
          jax.ShapeDtypeStruct((N, DH), jnp.float32),
          jax.ShapeDtypeStruct((N, DH), jnp.float32),
      ],
  )(num, num, den, den, w, ws, wd)


def _final_body(n0_ref, n1_ref, d0_ref, d1_ref, o_ref):
  r = _normalize(n0_ref[...], n1_ref[...], d0_ref[...], d1_ref[...])
  row = lax.broadcasted_iota(jnp.int32, (D, DH), 0)
  col = lax.broadcasted_iota(jnp.int32, (D, DH), 1)
  t = (row % DH == col).astype(jnp.float32) * (1.0 / H)
  o_ref[...] = jnp.dot(r, t, preferred_element_type=jnp.float32)


def _final(num, den):
  return pl.pallas_call(
      _final_body,
      grid=(NB,),
      in_specs=[
          pl.BlockSpec((BLK, D), lambda i: (i, 0)),
          pl.BlockSpec((BLK, D), lambda i: (i + NB, 0)),
          pl.BlockSpec((BLK, DH), lambda i: (i, 0)),
          pl.BlockSpec((BLK, DH), lambda i: (i + NB, 0)),
      ],
      out_specs=pl.BlockSpec((BLK, DH), lambda i: (i, 0)),
      out_shape=jax.ShapeDtypeStruct((N, DH), jnp.float32),
  )(num, num, den, den)


def _prep_weights(W, a):
  wf = W.transpose(1, 0, 2).reshape(W.shape[1], D)
  ws = jnp.einsum('hdk,hk->dh', W, a[:, :DH])
  wd = jnp.einsum('hdk,hk->dh', W, a[:, DH:])
  pad = jnp.zeros((W.shape[1], DH - H), jnp.float32)
  return wf, jnp.concatenate([ws, pad], 1), jnp.concatenate([wd, pad], 1)


def kernel(x, adj, W1, a1, W2, a2):
  idx = jnp.stack([adj[0].reshape(NCHUNK, CH), adj[1].reshape(NCHUNK, CH)],
                  axis=1)
  w1f, ws1, wd1 = _prep_weights(W1, a1)
  w2f, ws2, wd2 = _prep_weights(W2, a2)

  h1, fs1, fd1 = _proj(x, w1f, ws1, wd1)
  num1, den1 = _edge_pass(idx, h1, fs1, fd1)
  h2, fs2, fd2 = _mid(num1, den1, w2f, ws2, wd2)
  num2, den2 = _edge_pass(idx, h2, fs2, fd2)
  return _final(num2, den2)
